# baseline (speedup 1.0000x reference)
import functools

import jax
import jax.numpy as jnp
from jax.experimental import pallas as pl

D_NUM, T_NUM, D_DIM, T_DIM = 4096, 4096, 512, 512
UNITS = 256
K = 32
BLOCK_ROWS = 256
SW = 256
NC = T_NUM // 128


def _proj_kernel(hd_ref, w1_ref, ht_ref, w2_ref, o1_ref, o2_ref):
    o1_ref[...] = jnp.dot(hd_ref[...], w1_ref[...])
    o2_ref[...] = jnp.dot(ht_ref[...], w2_ref[...])


def _sim_kernel(hd_ref, ht_ref, l2_ref, sl_ref, bd_ref, r_ref, rf_ref):
    B = BLOCK_ROWS
    T = T_NUM
    S = jax.lax.dot_general(hd_ref[...], ht_ref[...],
                            (((1,), (1,)), ((), ())))
    R = jax.nn.sigmoid(S)
    r_ref[...] = R

    bits = jax.lax.bitcast_convert_type(R, jnp.int32)

    cmax = jnp.max(R.reshape(B, NC, 128), axis=2)
    lo0 = jax.lax.bitcast_convert_type(
        jnp.min(cmax, axis=1, keepdims=True), jnp.int32)
    hi0 = jax.lax.bitcast_convert_type(
        jnp.max(cmax, axis=1, keepdims=True), jnp.int32) + 1

    def cond(carry):
        lo, hi = carry
        return jnp.any(hi - lo > 1)

    def body(carry):
        lo, hi = carry
        mid = jax.lax.shift_right_logical(lo + hi, 1)
        cnt = jnp.sum(jnp.where(bits >= mid, 1.0, 0.0), axis=1, keepdims=True)
        ge = cnt >= K
        return jnp.where(ge, mid, lo), jnp.where(ge, hi, mid)

    t, _ = jax.lax.while_loop(cond, body, (lo0, hi0))

    gt = bits > t
    eq = bits == t
    n_gt = jnp.sum(jnp.where(gt, 1.0, 0.0), axis=1, keepdims=True)
    need = K - n_gt

    eqb = jnp.where(eq, 1.0, 0.0).astype(jnp.bfloat16)
    ptot = jax.lax.dot_general(eqb, bd_ref[...], (((1,), (0,)), ((), ())),
                               preferred_element_type=jnp.float32)
    coarse = jax.lax.dot_general(ptot.astype(jnp.bfloat16), sl_ref[...],
                                 (((1,), (0,)), ((), ())),
                                 preferred_element_type=jnp.float32)
    coarse = jnp.minimum(coarse, 64.0)

    l2 = l2_ref[...]
    for c in range(T // SW):
        sl_lo, sl_hi = c * SW, (c + 1) * SW
        P_c = jax.lax.dot_general(
            jax.lax.slice_in_dim(eqb, sl_lo, sl_hi, axis=1), l2,
            (((1,), (0,)), ((), ())), preferred_element_type=jnp.float32)
        offs_c = jnp.concatenate(
            [jnp.broadcast_to(
                jax.lax.slice_in_dim(coarse, 2 * c + h, 2 * c + h + 1, axis=1),
                (B, 128)) for h in (0, 1)], axis=1)
        eq_c = jax.lax.slice_in_dim(eq, sl_lo, sl_hi, axis=1)
        gt_c = jax.lax.slice_in_dim(gt, sl_lo, sl_hi, axis=1)
        R_c = jax.lax.slice_in_dim(R, sl_lo, sl_hi, axis=1)
        keep_c = gt_c | (eq_c & ((P_c + offs_c) <= need))
        rf_ref[:, sl_lo:sl_hi] = jnp.where(keep_c, R_c, 0.0)


def _constants():
    ir = jax.lax.broadcasted_iota(jnp.int32, (SW, SW), 0)
    ic = jax.lax.broadcasted_iota(jnp.int32, (SW, SW), 1)
    L2 = jnp.where((ir <= ic) & ((ir >> 7) == (ic >> 7)), 1.0, 0.0
                   ).astype(jnp.bfloat16)
    ia = jax.lax.broadcasted_iota(jnp.int32, (NC, NC), 0)
    ib = jax.lax.broadcasted_iota(jnp.int32, (NC, NC), 1)
    SL = jnp.where(ia < ib, 1.0, 0.0).astype(jnp.bfloat16)
    ro = jax.lax.broadcasted_iota(jnp.int32, (T_NUM, NC), 0)
    co = jax.lax.broadcasted_iota(jnp.int32, (T_NUM, NC), 1)
    BD = jnp.where((ro >> 7) == co, 1.0, 0.0).astype(jnp.bfloat16)
    return L2, SL, BD


@functools.partial(jax.jit)
def kernel(H_d, H_t, W1, W2):
    Hd, Ht = pl.pallas_call(
        _proj_kernel,
        out_shape=(jax.ShapeDtypeStruct((D_NUM, UNITS), jnp.float32),
                   jax.ShapeDtypeStruct((T_NUM, UNITS), jnp.float32)),
    )(H_d, W1, H_t, W2)

    L2, SL, BD = _constants()

    grid = D_NUM // BLOCK_ROWS
    R, R_flt = pl.pallas_call(
        _sim_kernel,
        grid=(grid,),
        in_specs=[
            pl.BlockSpec((BLOCK_ROWS, UNITS), lambda i: (i, 0)),
            pl.BlockSpec((T_NUM, UNITS), lambda i: (0, 0)),
            pl.BlockSpec((SW, SW), lambda i: (0, 0)),
            pl.BlockSpec((NC, NC), lambda i: (0, 0)),
            pl.BlockSpec((T_NUM, NC), lambda i: (0, 0)),
        ],
        out_specs=[
            pl.BlockSpec((BLOCK_ROWS, T_NUM), lambda i: (i, 0)),
            pl.BlockSpec((BLOCK_ROWS, T_NUM), lambda i: (i, 0)),
        ],
        out_shape=(jax.ShapeDtypeStruct((D_NUM, T_NUM), jnp.float32),
                   jax.ShapeDtypeStruct((D_NUM, T_NUM), jnp.float32)),
    )(Hd, Ht, L2, SL, BD)

    z = jnp.zeros(())
    return (R, R_flt, z, z, z, z)


# speedup vs baseline: 1.0048x; 1.0048x over previous
import functools

import jax
import jax.numpy as jnp
from jax.experimental import pallas as pl

D_NUM, T_NUM, D_DIM, T_DIM = 4096, 4096, 512, 512
UNITS = 256
K = 32
BLOCK_ROWS = 256
SW = 256
NC = T_NUM // 128


def _proj_kernel(hd_ref, w1_ref, ht_ref, w2_ref, o1_ref, o2_ref):
    o1_ref[...] = jnp.dot(hd_ref[...], w1_ref[...])
    o2_ref[...] = jnp.dot(ht_ref[...], w2_ref[...])


def _sim_kernel(hd_ref, ht_ref, l2_ref, sl_ref, bd_ref, r_ref, rf_ref):
    B = BLOCK_ROWS
    T = T_NUM
    S = jax.lax.dot_general(hd_ref[...], ht_ref[...],
                            (((1,), (1,)), ((), ())))
    R = jax.nn.sigmoid(S)
    r_ref[...] = R

    cmax = jnp.max(R.reshape(B, NC, 128), axis=2)
    lo0 = jax.lax.bitcast_convert_type(
        jnp.min(cmax, axis=1, keepdims=True), jnp.int32)
    hi0 = jax.lax.bitcast_convert_type(
        jnp.max(cmax, axis=1, keepdims=True), jnp.int32) + 1

    def cond(carry):
        lo, hi = carry
        return jnp.any(hi - lo > 1)

    def body(carry):
        lo, hi = carry
        mid = jax.lax.shift_right_logical(lo + hi, 1)
        midf = jax.lax.bitcast_convert_type(mid, jnp.float32)
        cnt = jnp.sum(jnp.where(R >= midf, 1.0, 0.0), axis=1, keepdims=True)
        ge = cnt >= K
        return jnp.where(ge, mid, lo), jnp.where(ge, hi, mid)

    t, _ = jax.lax.while_loop(cond, body, (lo0, hi0))
    tv = jax.lax.bitcast_convert_type(t, jnp.float32)

    gt = R > tv
    eq = R == tv
    n_gt = jnp.sum(jnp.where(gt, 1.0, 0.0), axis=1, keepdims=True)
    need = K - n_gt

    eqb = jnp.where(eq, 1.0, 0.0).astype(jnp.bfloat16)
    ptot = jax.lax.dot_general(eqb, bd_ref[...], (((1,), (0,)), ((), ())),
                               preferred_element_type=jnp.float32)
    coarse = jax.lax.dot_general(ptot.astype(jnp.bfloat16), sl_ref[...],
                                 (((1,), (0,)), ((), ())),
                                 preferred_element_type=jnp.float32)
    coarse = jnp.minimum(coarse, 64.0)

    thr = need - coarse

    l2 = l2_ref[...]
    for c in range(T // SW):
        sl_lo, sl_hi = c * SW, (c + 1) * SW
        P_c = jax.lax.dot_general(
            jax.lax.slice_in_dim(eqb, sl_lo, sl_hi, axis=1), l2,
            (((1,), (0,)), ((), ())), preferred_element_type=jnp.float32)
        for h in (0, 1):
            a, b = sl_lo + h * 128, sl_lo + (h + 1) * 128
            thr_h = jax.lax.slice_in_dim(thr, 2 * c + h, 2 * c + h + 1, axis=1)
            P_h = jax.lax.slice_in_dim(P_c, h * 128, (h + 1) * 128, axis=1)
            eq_h = jax.lax.slice_in_dim(eq, a, b, axis=1)
            gt_h = jax.lax.slice_in_dim(gt, a, b, axis=1)
            R_h = jax.lax.slice_in_dim(R, a, b, axis=1)
            keep_h = gt_h | (eq_h & (P_h <= thr_h))
            rf_ref[:, a:b] = jnp.where(keep_h, R_h, 0.0)


def _constants():
    ir = jax.lax.broadcasted_iota(jnp.int32, (SW, SW), 0)
    ic = jax.lax.broadcasted_iota(jnp.int32, (SW, SW), 1)
    L2 = jnp.where((ir <= ic) & ((ir >> 7) == (ic >> 7)), 1.0, 0.0
                   ).astype(jnp.bfloat16)
    ia = jax.lax.broadcasted_iota(jnp.int32, (NC, NC), 0)
    ib = jax.lax.broadcasted_iota(jnp.int32, (NC, NC), 1)
    SL = jnp.where(ia < ib, 1.0, 0.0).astype(jnp.bfloat16)
    ro = jax.lax.broadcasted_iota(jnp.int32, (T_NUM, NC), 0)
    co = jax.lax.broadcasted_iota(jnp.int32, (T_NUM, NC), 1)
    BD = jnp.where((ro >> 7) == co, 1.0, 0.0).astype(jnp.bfloat16)
    return L2, SL, BD


@functools.partial(jax.jit)
def kernel(H_d, H_t, W1, W2):
    Hd, Ht = pl.pallas_call(
        _proj_kernel,
        out_shape=(jax.ShapeDtypeStruct((D_NUM, UNITS), jnp.float32),
                   jax.ShapeDtypeStruct((T_NUM, UNITS), jnp.float32)),
    )(H_d, W1, H_t, W2)

    L2, SL, BD = _constants()

    grid = D_NUM // BLOCK_ROWS
    R, R_flt = pl.pallas_call(
        _sim_kernel,
        grid=(grid,),
        in_specs=[
            pl.BlockSpec((BLOCK_ROWS, UNITS), lambda i: (i, 0)),
            pl.BlockSpec((T_NUM, UNITS), lambda i: (0, 0)),
            pl.BlockSpec((SW, SW), lambda i: (0, 0)),
            pl.BlockSpec((NC, NC), lambda i: (0, 0)),
            pl.BlockSpec((T_NUM, NC), lambda i: (0, 0)),
        ],
        out_specs=[
            pl.BlockSpec((BLOCK_ROWS, T_NUM), lambda i: (i, 0)),
            pl.BlockSpec((BLOCK_ROWS, T_NUM), lambda i: (i, 0)),
        ],
        out_shape=(jax.ShapeDtypeStruct((D_NUM, T_NUM), jnp.float32),
                   jax.ShapeDtypeStruct((D_NUM, T_NUM), jnp.float32)),
    )(Hd, Ht, L2, SL, BD)

    z = jnp.zeros(())
    return (R, R_flt, z, z, z, z)


# speedup vs baseline: 1.0442x; 1.0392x over previous
import functools

import jax
import jax.numpy as jnp
from jax.experimental import pallas as pl

D_NUM, T_NUM, D_DIM, T_DIM = 4096, 4096, 512, 512
UNITS = 256
K = 32
BLOCK_ROWS = 256
SW = 256
NC = T_NUM // 128


def _proj_kernel(hd_ref, w1_ref, ht_ref, w2_ref, o1_ref, o2_ref):
    o1_ref[...] = jnp.dot(hd_ref[...], w1_ref[...])
    o2_ref[...] = jnp.dot(ht_ref[...], w2_ref[...])


def _sim_kernel(hd_ref, ht_ref, r_ref, rf_ref):
    B = BLOCK_ROWS
    T = T_NUM
    S = jax.lax.dot_general(hd_ref[...], ht_ref[...],
                            (((1,), (1,)), ((), ())))
    R = jax.nn.sigmoid(S)
    r_ref[...] = R

    cmax = jnp.max(R.reshape(B, NC, 128), axis=2)
    lo0 = jax.lax.bitcast_convert_type(
        jnp.min(cmax, axis=1, keepdims=True), jnp.int32)
    hi0 = jax.lax.bitcast_convert_type(
        jnp.max(cmax, axis=1, keepdims=True), jnp.int32) + 1

    def cond(carry):
        lo, hi = carry
        return jnp.any(hi - lo > 1)

    def body(carry):
        lo, hi = carry
        mid = jax.lax.shift_right_logical(lo + hi, 1)
        midf = jax.lax.bitcast_convert_type(mid, jnp.float32)
        cnt = jnp.sum(jnp.where(R >= midf, 1.0, 0.0), axis=1, keepdims=True)
        ge = cnt >= K
        return jnp.where(ge, mid, lo), jnp.where(ge, hi, mid)

    t, _ = jax.lax.while_loop(cond, body, (lo0, hi0))
    tv = jax.lax.bitcast_convert_type(t, jnp.float32)

    gt = R > tv
    eq = R == tv
    n_gt = jnp.sum(jnp.where(gt, 1.0, 0.0), axis=1, keepdims=True)
    need = K - n_gt

    eqb = jnp.where(eq, 1.0, 0.0).astype(jnp.bfloat16)

    ir = jax.lax.broadcasted_iota(jnp.int32, (SW, SW), 0)
    ic = jax.lax.broadcasted_iota(jnp.int32, (SW, SW), 1)
    l2 = jnp.where((ir <= ic) & ((ir >> 7) == (ic >> 7)), 1.0, 0.0
                   ).astype(jnp.bfloat16)
    ia = jax.lax.broadcasted_iota(jnp.int32, (NC, NC), 0)
    ib = jax.lax.broadcasted_iota(jnp.int32, (NC, NC), 1)
    slt = jnp.where(ia < ib, 1.0, 0.0).astype(jnp.bfloat16)

    Ps = []
    for c in range(T // SW):
        sl_lo, sl_hi = c * SW, (c + 1) * SW
        Ps.append(jax.lax.dot_general(
            jax.lax.slice_in_dim(eqb, sl_lo, sl_hi, axis=1), l2,
            (((1,), (0,)), ((), ())), preferred_element_type=jnp.float32))

    ptot = jnp.concatenate(
        [jax.lax.slice_in_dim(P_c, h * 128 + 127, h * 128 + 128, axis=1)
         for P_c in Ps for h in (0, 1)], axis=1)
    coarse = jax.lax.dot_general(ptot.astype(jnp.bfloat16), slt,
                                 (((1,), (0,)), ((), ())),
                                 preferred_element_type=jnp.float32)
    coarse = jnp.minimum(coarse, 64.0)

    thr = need - coarse

    for c in range(T // SW):
        sl_lo, sl_hi = c * SW, (c + 1) * SW
        P_c = Ps[c]
        for h in (0, 1):
            a, b = sl_lo + h * 128, sl_lo + (h + 1) * 128
            thr_h = jax.lax.slice_in_dim(thr, 2 * c + h, 2 * c + h + 1, axis=1)
            P_h = jax.lax.slice_in_dim(P_c, h * 128, (h + 1) * 128, axis=1)
            eq_h = jax.lax.slice_in_dim(eq, a, b, axis=1)
            gt_h = jax.lax.slice_in_dim(gt, a, b, axis=1)
            R_h = jax.lax.slice_in_dim(R, a, b, axis=1)
            keep_h = gt_h | (eq_h & (P_h <= thr_h))
            rf_ref[:, a:b] = jnp.where(keep_h, R_h, 0.0)


@functools.partial(jax.jit)
def kernel(H_d, H_t, W1, W2):
    Hd, Ht = pl.pallas_call(
        _proj_kernel,
        out_shape=(jax.ShapeDtypeStruct((D_NUM, UNITS), jnp.float32),
                   jax.ShapeDtypeStruct((T_NUM, UNITS), jnp.float32)),
    )(H_d, W1, H_t, W2)

    grid = D_NUM // BLOCK_ROWS
    R, R_flt = pl.pallas_call(
        _sim_kernel,
        grid=(grid,),
        in_specs=[
            pl.BlockSpec((BLOCK_ROWS, UNITS), lambda i: (i, 0)),
            pl.BlockSpec((T_NUM, UNITS), lambda i: (0, 0)),
        ],
        out_specs=[
            pl.BlockSpec((BLOCK_ROWS, T_NUM), lambda i: (i, 0)),
            pl.BlockSpec((BLOCK_ROWS, T_NUM), lambda i: (i, 0)),
        ],
        out_shape=(jax.ShapeDtypeStruct((D_NUM, T_NUM), jnp.float32),
                   jax.ShapeDtypeStruct((D_NUM, T_NUM), jnp.float32)),
    )(Hd, Ht)

    z = jnp.zeros(())
    return (R, R_flt, z, z, z, z)


# speedup vs baseline: 1.1035x; 1.0569x over previous
import functools

import jax
import jax.numpy as jnp
from jax.experimental import pallas as pl

D_NUM, T_NUM, D_DIM, T_DIM = 4096, 4096, 512, 512
UNITS = 256
K = 32
BLOCK_ROWS = 512
SW = 256
NC = T_NUM // 128


def _proj_kernel(hd_ref, w1_ref, ht_ref, w2_ref, o1_ref, o2_ref):
    o1_ref[...] = jnp.dot(hd_ref[...], w1_ref[...])
    o2_ref[...] = jnp.dot(ht_ref[...], w2_ref[...])


def _sim_kernel(hd_ref, ht_ref, r_ref, rf_ref):
    B = BLOCK_ROWS
    T = T_NUM
    S = jax.lax.dot_general(hd_ref[...], ht_ref[...],
                            (((1,), (1,)), ((), ())))
    R = jax.nn.sigmoid(S)
    r_ref[...] = R

    cmax = jnp.max(R.reshape(B, NC, 128), axis=2)
    lo0 = jax.lax.bitcast_convert_type(
        jnp.min(cmax, axis=1, keepdims=True), jnp.int32)
    hi0 = jax.lax.bitcast_convert_type(
        jnp.max(cmax, axis=1, keepdims=True), jnp.int32) + 1

    def cond(carry):
        lo, hi = carry
        return jnp.any(hi - lo > 1)

    def body(carry):
        lo, hi = carry
        mid = jax.lax.shift_right_logical(lo + hi, 1)
        midf = jax.lax.bitcast_convert_type(mid, jnp.float32)
        cnt = jnp.sum(jnp.where(R >= midf, 1.0, 0.0), axis=1, keepdims=True)
        ge = cnt >= K
        return jnp.where(ge, mid, lo), jnp.where(ge, hi, mid)

    t, _ = jax.lax.while_loop(cond, body, (lo0, hi0))
    tv = jax.lax.bitcast_convert_type(t, jnp.float32)

    gt = R > tv
    eq = R == tv
    n_gt = jnp.sum(jnp.where(gt, 1.0, 0.0), axis=1, keepdims=True)
    need = K - n_gt

    eqb = jnp.where(eq, 1.0, 0.0).astype(jnp.bfloat16)

    ir = jax.lax.broadcasted_iota(jnp.int32, (SW, SW), 0)
    ic = jax.lax.broadcasted_iota(jnp.int32, (SW, SW), 1)
    l2 = jnp.where((ir <= ic) & ((ir >> 7) == (ic >> 7)), 1.0, 0.0
                   ).astype(jnp.bfloat16)
    ia = jax.lax.broadcasted_iota(jnp.int32, (NC, NC), 0)
    ib = jax.lax.broadcasted_iota(jnp.int32, (NC, NC), 1)
    slt = jnp.where(ia < ib, 1.0, 0.0).astype(jnp.bfloat16)

    Ps = []
    for c in range(T // SW):
        sl_lo, sl_hi = c * SW, (c + 1) * SW
        Ps.append(jax.lax.dot_general(
            jax.lax.slice_in_dim(eqb, sl_lo, sl_hi, axis=1), l2,
            (((1,), (0,)), ((), ())), preferred_element_type=jnp.float32))

    ptot = jnp.concatenate(
        [jax.lax.slice_in_dim(P_c, h * 128 + 127, h * 128 + 128, axis=1)
         for P_c in Ps for h in (0, 1)], axis=1)
    coarse = jax.lax.dot_general(ptot.astype(jnp.bfloat16), slt,
                                 (((1,), (0,)), ((), ())),
                                 preferred_element_type=jnp.float32)
    coarse = jnp.minimum(coarse, 64.0)

    thr = need - coarse

    for c in range(T // SW):
        sl_lo, sl_hi = c * SW, (c + 1) * SW
        P_c = Ps[c]
        for h in (0, 1):
            a, b = sl_lo + h * 128, sl_lo + (h + 1) * 128
            thr_h = jax.lax.slice_in_dim(thr, 2 * c + h, 2 * c + h + 1, axis=1)
            P_h = jax.lax.slice_in_dim(P_c, h * 128, (h + 1) * 128, axis=1)
            eq_h = jax.lax.slice_in_dim(eq, a, b, axis=1)
            gt_h = jax.lax.slice_in_dim(gt, a, b, axis=1)
            R_h = jax.lax.slice_in_dim(R, a, b, axis=1)
            keep_h = gt_h | (eq_h & (P_h <= thr_h))
            rf_ref[:, a:b] = jnp.where(keep_h, R_h, 0.0)


@functools.partial(jax.jit)
def kernel(H_d, H_t, W1, W2):
    Hd, Ht = pl.pallas_call(
        _proj_kernel,
        out_shape=(jax.ShapeDtypeStruct((D_NUM, UNITS), jnp.float32),
                   jax.ShapeDtypeStruct((T_NUM, UNITS), jnp.float32)),
    )(H_d, W1, H_t, W2)

    grid = D_NUM // BLOCK_ROWS
    R, R_flt = pl.pallas_call(
        _sim_kernel,
        grid=(grid,),
        in_specs=[
            pl.BlockSpec((BLOCK_ROWS, UNITS), lambda i: (i, 0)),
            pl.BlockSpec((T_NUM, UNITS), lambda i: (0, 0)),
        ],
        out_specs=[
            pl.BlockSpec((BLOCK_ROWS, T_NUM), lambda i: (i, 0)),
            pl.BlockSpec((BLOCK_ROWS, T_NUM), lambda i: (i, 0)),
        ],
        out_shape=(jax.ShapeDtypeStruct((D_NUM, T_NUM), jnp.float32),
                   jax.ShapeDtypeStruct((D_NUM, T_NUM), jnp.float32)),
    )(Hd, Ht)

    z = jnp.zeros(())
    return (R, R_flt, z, z, z, z)


# speedup vs baseline: 1.1694x; 1.0596x over previous
import functools

import jax
import jax.numpy as jnp
from jax.experimental import pallas as pl
from jax.experimental.pallas import tpu as pltpu

D_NUM, T_NUM, D_DIM, T_DIM = 4096, 4096, 512, 512
UNITS = 256
K = 32
BLOCK_ROWS = 512
SW = 256
NC = T_NUM // 128


def _proj_kernel(ht_ref, w2_ref, o_ref):
    o_ref[...] = jnp.dot(ht_ref[...], w2_ref[...])


def _sim_kernel(hd_raw_ref, w1_ref, ht_ref, r_ref, rf_ref):
    B = BLOCK_ROWS
    T = T_NUM

    hd = jnp.dot(hd_raw_ref[...], w1_ref[...])
    S = jax.lax.dot_general(hd, ht_ref[...], (((1,), (1,)), ((), ())))
    R = jax.nn.sigmoid(S)
    r_ref[...] = R

    cmax = jnp.max(R.reshape(B, NC, 128), axis=2)
    lo0 = jax.lax.bitcast_convert_type(
        jnp.min(cmax, axis=1, keepdims=True), jnp.int32)
    hi0 = jax.lax.bitcast_convert_type(
        jnp.max(cmax, axis=1, keepdims=True), jnp.int32) + 1

    def cond(carry):
        lo, hi = carry
        return jnp.any(hi - lo > 1)

    def body(carry):
        lo, hi = carry
        mid = jax.lax.shift_right_logical(lo + hi, 1)
        midf = jax.lax.bitcast_convert_type(mid, jnp.float32)
        cnt = jnp.sum(jnp.where(R >= midf, 1.0, 0.0), axis=1, keepdims=True)
        ge = cnt >= K
        return jnp.where(ge, mid, lo), jnp.where(ge, hi, mid)

    t, _ = jax.lax.while_loop(cond, body, (lo0, hi0))
    tv = jax.lax.bitcast_convert_type(t, jnp.float32)

    gt = R > tv
    eq = R == tv
    n_gt = jnp.sum(jnp.where(gt, 1.0, 0.0), axis=1, keepdims=True)
    need = K - n_gt

    eqb = jnp.where(eq, 1.0, 0.0).astype(jnp.bfloat16)

    ir = jax.lax.broadcasted_iota(jnp.int32, (SW, SW), 0)
    ic = jax.lax.broadcasted_iota(jnp.int32, (SW, SW), 1)
    l2 = jnp.where((ir <= ic) & ((ir >> 7) == (ic >> 7)), 1.0, 0.0
                   ).astype(jnp.bfloat16)
    ia = jax.lax.broadcasted_iota(jnp.int32, (NC, NC), 0)
    ib = jax.lax.broadcasted_iota(jnp.int32, (NC, NC), 1)
    slt = jnp.where(ia < ib, 1.0, 0.0).astype(jnp.bfloat16)

    Ps = []
    for c in range(T // SW):
        sl_lo, sl_hi = c * SW, (c + 1) * SW
        Ps.append(jax.lax.dot_general(
            jax.lax.slice_in_dim(eqb, sl_lo, sl_hi, axis=1), l2,
            (((1,), (0,)), ((), ())), preferred_element_type=jnp.float32))

    ptot = jnp.concatenate(
        [jax.lax.slice_in_dim(P_c, h * 128 + 127, h * 128 + 128, axis=1)
         for P_c in Ps for h in (0, 1)], axis=1)
    coarse = jax.lax.dot_general(ptot.astype(jnp.bfloat16), slt,
                                 (((1,), (0,)), ((), ())),
                                 preferred_element_type=jnp.float32)
    coarse = jnp.minimum(coarse, 64.0)

    thr = need - coarse

    for c in range(T // SW):
        sl_lo, sl_hi = c * SW, (c + 1) * SW
        P_c = Ps[c]
        for h in (0, 1):
            a, b = sl_lo + h * 128, sl_lo + (h + 1) * 128
            thr_h = jax.lax.slice_in_dim(thr, 2 * c + h, 2 * c + h + 1, axis=1)
            P_h = jax.lax.slice_in_dim(P_c, h * 128, (h + 1) * 128, axis=1)
            eq_h = jax.lax.slice_in_dim(eq, a, b, axis=1)
            gt_h = jax.lax.slice_in_dim(gt, a, b, axis=1)
            R_h = jax.lax.slice_in_dim(R, a, b, axis=1)
            keep_h = gt_h | (eq_h & (P_h <= thr_h))
            rf_ref[:, a:b] = jnp.where(keep_h, R_h, 0.0)


@functools.partial(jax.jit)
def kernel(H_d, H_t, W1, W2):
    Ht = pl.pallas_call(
        _proj_kernel,
        out_shape=jax.ShapeDtypeStruct((T_NUM, UNITS), jnp.float32),
    )(H_t, W2)

    grid = D_NUM // BLOCK_ROWS
    R, R_flt = pl.pallas_call(
        _sim_kernel,
        grid=(grid,),
        in_specs=[
            pl.BlockSpec((BLOCK_ROWS, D_DIM), lambda i: (i, 0)),
            pl.BlockSpec((D_DIM, UNITS), lambda i: (0, 0)),
            pl.BlockSpec((T_NUM, UNITS), lambda i: (0, 0)),
        ],
        out_specs=[
            pl.BlockSpec((BLOCK_ROWS, T_NUM), lambda i: (i, 0)),
            pl.BlockSpec((BLOCK_ROWS, T_NUM), lambda i: (i, 0)),
        ],
        out_shape=(jax.ShapeDtypeStruct((D_NUM, T_NUM), jnp.float32),
                   jax.ShapeDtypeStruct((D_NUM, T_NUM), jnp.float32)),
    )(H_d, W1, Ht)

    z = jnp.zeros(())
    return (R, R_flt, z, z, z, z)


# speedup vs baseline: 1.2416x; 1.0618x over previous
import functools

import jax
import jax.numpy as jnp
from jax.experimental import pallas as pl
from jax.experimental.pallas import tpu as pltpu

D_NUM, T_NUM, D_DIM, T_DIM = 4096, 4096, 512, 512
UNITS = 256
K = 32
BLOCK_ROWS = 512
SW = 256
NC = T_NUM // 128


def _proj_kernel(ht_ref, w2_ref, o_ref):
    o_ref[...] = jnp.dot(ht_ref[...], w2_ref[...])


def _sim_kernel(hd_raw_ref, w1_ref, ht_ref, r_ref, rf_ref):
    B = BLOCK_ROWS
    T = T_NUM

    hd = jnp.dot(hd_raw_ref[...], w1_ref[...])
    S = jax.lax.dot_general(hd, ht_ref[...], (((1,), (1,)), ((), ())))
    R = jax.nn.sigmoid(S)
    r_ref[...] = R

    cmaxs = [jnp.max(jax.lax.slice_in_dim(R, c * 128, (c + 1) * 128, axis=1),
                     axis=1, keepdims=True) for c in range(NC)]
    lo0 = jax.lax.bitcast_convert_type(
        functools.reduce(jnp.minimum, cmaxs), jnp.int32)
    hi0 = jax.lax.bitcast_convert_type(
        functools.reduce(jnp.maximum, cmaxs), jnp.int32) + 1

    def cond(carry):
        lo, hi = carry
        return jnp.any(hi - lo > 1)

    def body(carry):
        lo, hi = carry
        mid = jax.lax.shift_right_logical(lo + hi, 1)
        midf = jax.lax.bitcast_convert_type(mid, jnp.float32)
        cnt = jnp.sum(jnp.where(R >= midf, 1.0, 0.0), axis=1, keepdims=True)
        ge = cnt >= K
        return jnp.where(ge, mid, lo), jnp.where(ge, hi, mid)

    t, _ = jax.lax.while_loop(cond, body, (lo0, hi0))
    tv = jax.lax.bitcast_convert_type(t, jnp.float32)

    n_gt = jnp.sum(jnp.where(R > tv, 1.0, 0.0), axis=1, keepdims=True)
    need = K - n_gt

    eqb = jnp.where(R == tv, 1.0, 0.0).astype(jnp.bfloat16)

    ir = jax.lax.broadcasted_iota(jnp.int32, (SW, SW), 0)
    ic = jax.lax.broadcasted_iota(jnp.int32, (SW, SW), 1)
    l2 = jnp.where((ir <= ic) & ((ir >> 7) == (ic >> 7)), 1.0, 0.0
                   ).astype(jnp.bfloat16)
    ia = jax.lax.broadcasted_iota(jnp.int32, (NC, NC), 0)
    ib = jax.lax.broadcasted_iota(jnp.int32, (NC, NC), 1)
    slt = jnp.where(ia < ib, 1.0, 0.0).astype(jnp.bfloat16)

    Ps = []
    for c in range(T // SW):
        sl_lo, sl_hi = c * SW, (c + 1) * SW
        Ps.append(jax.lax.dot_general(
            jax.lax.slice_in_dim(eqb, sl_lo, sl_hi, axis=1), l2,
            (((1,), (0,)), ((), ())), preferred_element_type=jnp.float32))

    ptot = jnp.concatenate(
        [jax.lax.slice_in_dim(P_c, h * 128 + 127, h * 128 + 128, axis=1)
         for P_c in Ps for h in (0, 1)], axis=1)
    coarse = jax.lax.dot_general(ptot.astype(jnp.bfloat16), slt,
                                 (((1,), (0,)), ((), ())),
                                 preferred_element_type=jnp.float32)
    coarse = jnp.minimum(coarse, 64.0)

    thr = need - coarse

    for c in range(T // SW):
        sl_lo, sl_hi = c * SW, (c + 1) * SW
        P_c = Ps[c]
        for h in (0, 1):
            a, b = sl_lo + h * 128, sl_lo + (h + 1) * 128
            thr_h = jax.lax.slice_in_dim(thr, 2 * c + h, 2 * c + h + 1, axis=1)
            P_h = jax.lax.slice_in_dim(P_c, h * 128, (h + 1) * 128, axis=1)
            R_h = jax.lax.slice_in_dim(R, a, b, axis=1)
            keep_h = (R_h > tv) | ((R_h == tv) & (P_h <= thr_h))
            rf_ref[:, a:b] = jnp.where(keep_h, R_h, 0.0)


@functools.partial(jax.jit)
def kernel(H_d, H_t, W1, W2):
    Ht = pl.pallas_call(
        _proj_kernel,
        out_shape=jax.ShapeDtypeStruct((T_NUM, UNITS), jnp.float32),
    )(H_t, W2)

    grid = D_NUM // BLOCK_ROWS
    R, R_flt = pl.pallas_call(
        _sim_kernel,
        grid=(grid,),
        in_specs=[
            pl.BlockSpec((BLOCK_ROWS, D_DIM), lambda i: (i, 0)),
            pl.BlockSpec((D_DIM, UNITS), lambda i: (0, 0)),
            pl.BlockSpec((T_NUM, UNITS), lambda i: (0, 0)),
        ],
        out_specs=[
            pl.BlockSpec((BLOCK_ROWS, T_NUM), lambda i: (i, 0)),
            pl.BlockSpec((BLOCK_ROWS, T_NUM), lambda i: (i, 0)),
        ],
        out_shape=(jax.ShapeDtypeStruct((D_NUM, T_NUM), jnp.float32),
                   jax.ShapeDtypeStruct((D_NUM, T_NUM), jnp.float32)),
    )(H_d, W1, Ht)

    z = jnp.zeros(())
    return (R, R_flt, z, z, z, z)


# speedup vs baseline: 1.2464x; 1.0039x over previous
import functools

import jax
import jax.numpy as jnp
from jax.experimental import pallas as pl

D_NUM, T_NUM, D_DIM, T_DIM = 4096, 4096, 512, 512
UNITS = 256
K = 32
BLOCK_ROWS = 512
SW = 256
NC = T_NUM // 128


def _proj_kernel(ht_ref, w2_ref, o_ref):
    o_ref[...] = jnp.dot(ht_ref[...], w2_ref[...])


def _sim_kernel(hd_raw_ref, w1_ref, ht_ref, r_ref, rf_ref):
    B = BLOCK_ROWS
    T = T_NUM

    hd = jnp.dot(hd_raw_ref[...], w1_ref[...])
    S = jax.lax.dot_general(hd, ht_ref[...], (((1,), (1,)), ((), ())))
    R = jax.nn.sigmoid(S)
    r_ref[...] = R

    cmaxs = [jnp.max(jax.lax.slice_in_dim(R, c * 128, (c + 1) * 128, axis=1),
                     axis=1, keepdims=True) for c in range(NC)]
    lo0 = jax.lax.bitcast_convert_type(
        functools.reduce(jnp.minimum, cmaxs), jnp.int32)
    hi0 = jax.lax.bitcast_convert_type(
        functools.reduce(jnp.maximum, cmaxs), jnp.int32) + 1

    def cond(carry):
        lo, hi = carry
        return jnp.any(hi - lo > 1)

    def body(carry):
        lo, hi = carry
        mid = jax.lax.shift_right_logical(lo + hi, 1)
        midf = jax.lax.bitcast_convert_type(mid, jnp.float32)
        cnt = jnp.sum(jnp.where(R >= midf, 1.0, 0.0), axis=1, keepdims=True)
        ge = cnt >= K
        return jnp.where(ge, mid, lo), jnp.where(ge, hi, mid)

    t, _ = jax.lax.while_loop(cond, body, (lo0, hi0))
    tv = jax.lax.bitcast_convert_type(t, jnp.float32)

    n_gt = jnp.sum(jnp.where(R > tv, 1.0, 0.0), axis=1, keepdims=True)
    need = K - n_gt

    eqb = jnp.where(R == tv, 1.0, 0.0).astype(jnp.bfloat16)

    ir = jax.lax.broadcasted_iota(jnp.int32, (SW, SW), 0)
    ic = jax.lax.broadcasted_iota(jnp.int32, (SW, SW), 1)
    l2 = jnp.where((ir <= ic) & ((ir >> 7) == (ic >> 7)), 1.0, 0.0
                   ).astype(jnp.bfloat16)
    ia = jax.lax.broadcasted_iota(jnp.int32, (NC, NC), 0)
    ib = jax.lax.broadcasted_iota(jnp.int32, (NC, NC), 1)
    slt = jnp.where(ia < ib, 1.0, 0.0).astype(jnp.bfloat16)

    Ps = []
    for c in range(T // SW):
        sl_lo, sl_hi = c * SW, (c + 1) * SW
        Ps.append(jax.lax.dot_general(
            jax.lax.slice_in_dim(eqb, sl_lo, sl_hi, axis=1), l2,
            (((1,), (0,)), ((), ())), preferred_element_type=jnp.float32))

    ptot = jnp.concatenate(
        [jax.lax.slice_in_dim(P_c, h * 128 + 127, h * 128 + 128, axis=1)
         for P_c in Ps for h in (0, 1)], axis=1)
    coarse = jax.lax.dot_general(ptot.astype(jnp.bfloat16), slt,
                                 (((1,), (0,)), ((), ())),
                                 preferred_element_type=jnp.float32)
    coarse = jnp.minimum(coarse, 64.0)

    thr = need - coarse

    for c in range(T // SW):
        sl_lo, sl_hi = c * SW, (c + 1) * SW
        P_c = Ps[c]
        for h in (0, 1):
            a, b = sl_lo + h * 128, sl_lo + (h + 1) * 128
            thr_h = jax.lax.slice_in_dim(thr, 2 * c + h, 2 * c + h + 1, axis=1)
            P_h = jax.lax.slice_in_dim(P_c, h * 128, (h + 1) * 128, axis=1)
            R_h = jax.lax.slice_in_dim(R, a, b, axis=1)
            keep_h = (R_h > tv) | ((R_h == tv) & (P_h <= thr_h))
            rf_ref[:, a:b] = jnp.where(keep_h, R_h, 0.0)


@functools.partial(jax.jit)
def kernel(H_d, H_t, W1, W2):
    Ht = pl.pallas_call(
        _proj_kernel,
        out_shape=jax.ShapeDtypeStruct((T_NUM, UNITS), jnp.float32),
    )(H_t, W2)

    grid = D_NUM // BLOCK_ROWS
    R, R_flt = pl.pallas_call(
        _sim_kernel,
        grid=(grid,),
        in_specs=[
            pl.BlockSpec((BLOCK_ROWS, D_DIM), lambda i: (i, 0)),
            pl.BlockSpec((D_DIM, UNITS), lambda i: (0, 0)),
            pl.BlockSpec((T_NUM, UNITS), lambda i: (0, 0)),
        ],
        out_specs=[
            pl.BlockSpec((BLOCK_ROWS, T_NUM), lambda i: (i, 0)),
            pl.BlockSpec((BLOCK_ROWS, T_NUM), lambda i: (i, 0)),
        ],
        out_shape=(jax.ShapeDtypeStruct((D_NUM, T_NUM), jnp.float32),
                   jax.ShapeDtypeStruct((D_NUM, T_NUM), jnp.float32)),
    )(H_d, W1, Ht)

    z = jnp.zeros(())
    return (R, R_flt, z, z, z, z)
